# Initial kernel scaffold; baseline (speedup 1.0000x reference)
import jax
import jax.numpy as jnp
from jax.experimental import pallas as pl


def kernel(x, pos, batch, focal_points, fa_w1, fa_b1, fa_w2, fa_b2, pa_w1, pa_b1, pa_w2, pa_b2):
    raise NotImplementedError("write your pallas kernel here")


# speedup vs baseline: 2.2633x; 2.2633x over previous
import functools

import jax
import jax.numpy as jnp
from jax import lax
from jax.experimental import pallas as pl
from jax.experimental.pallas import tpu as pltpu
from jax.experimental.pallas import tpu_sc as plsc

NUM_SEGMENTS = 8
K = 16
ATT_DIM = 32
D_COMB = 131
D_PAD = 256
LANE_BATCH = 131
ROW_TILE = 1000
Q_TILE = 128


def _combined_tile(x, pos, batchf, w):
    (fa_w1, fa_b1, fa_w2, fa_b2, pa_w1, pa_b1, pa_w2, pa_b2) = w
    l32 = lax.broadcasted_iota(jnp.int32, (1, 32), 1)
    band = (l32 // 2).astype(jnp.float32)
    freqs = 1.0 + band * jnp.float32(9.0 / 63.0)
    parity = l32 % 2
    xf = pos[:, 0:1] * freqs
    enc = jnp.where(parity == 0, jnp.sin(xf), jnp.cos(xf))

    fe = jnp.concatenate([x, enc], axis=1)
    pe = jnp.concatenate([pos, enc], axis=1)

    h1 = jnp.maximum(jnp.dot(fe, fa_w1, preferred_element_type=jnp.float32)
                     + fa_b1, 0.0)
    fw = jnp.dot(h1, fa_w2, preferred_element_type=jnp.float32) + fa_b2
    h2 = jnp.maximum(jnp.dot(pe, pa_w1, preferred_element_type=jnp.float32)
                     + pa_b1, 0.0)
    pw = jnp.dot(h2, pa_w2, preferred_element_type=jnp.float32) + pa_b2

    m = jnp.maximum(fw, pw)
    ef = jnp.exp(fw - m)
    ep = jnp.exp(pw - m)
    denom = ef + ep
    return jnp.concatenate([pe * (ep / denom), fe * (ef / denom)], axis=1)


def _stage1_kernel(x_ref, pos_ref, batchf_ref,
                   fa_w1_ref, fa_b1_ref, fa_w2_ref, fa_b2_ref,
                   pa_w1_ref, pa_b1_ref, pa_w2_ref, pa_b2_ref,
                   comb_ref, cnt_ref, sum_ref, sq_ref):
    batchf = batchf_ref[...]
    w = (fa_w1_ref[...], fa_b1_ref[...], fa_w2_ref[...], fa_b2_ref[...],
         pa_w1_ref[...], pa_b1_ref[...], pa_w2_ref[...], pa_b2_ref[...])
    combined = _combined_tile(x_ref[...], pos_ref[...], batchf, w)
    comb_ref[...] = combined

    @pl.when(pl.program_id(0) == 0)
    def _init():
        cnt_ref[...] = jnp.zeros_like(cnt_ref)
        sum_ref[...] = jnp.zeros_like(sum_ref)
        sq_ref[...] = jnp.zeros_like(sq_ref)

    cnts = []
    sums = []
    sqs = []
    for s in range(NUM_SEGMENTS):
        msk = (batchf == float(s))
        cm = jnp.where(msk, combined, 0.0)
        cnts.append(jnp.sum(jnp.where(msk, 1.0, 0.0)).reshape(1, 1))
        sums.append(jnp.sum(cm, axis=0, keepdims=True))
        sqs.append(jnp.sum(cm * cm, axis=0, keepdims=True))
    cnt_ref[...] += jnp.concatenate(cnts, axis=0)
    sum_ref[...] += jnp.concatenate(sums, axis=0)
    sq_ref[...] += jnp.concatenate(sqs, axis=0)


def _stage2_kernel(comb_ref, batchf_ref, cnt_ref, sum_ref, sq_ref,
                   table_ref, cn_ref):
    combined = comb_ref[...]
    batchf = batchf_ref[...]
    t = combined.shape[0]

    mean_full = jnp.zeros_like(combined)
    std_full = jnp.zeros_like(combined)
    for s in range(NUM_SEGMENTS):
        cnt = cnt_ref[s, 0]
        ssum = sum_ref[s:s + 1, :]
        sqsum = sq_ref[s:s + 1, :]
        cnt_c = jnp.maximum(cnt, 1.0)
        mean = ssum / cnt_c
        var = (sqsum - ssum * ssum / cnt_c) / jnp.maximum(cnt_c - 1.0, 1.0)
        std = jnp.sqrt(jnp.clip(var, 0.0, None))
        msk = (batchf == float(s))
        mean_full = jnp.where(msk, mean, mean_full)
        std_full = jnp.where(msk, std, std_full)
    normed = (combined - mean_full) / (std_full + 1e-8)

    cn_ref[...] = jnp.sum(normed * normed, axis=1, keepdims=True)
    pad = jnp.zeros((t, D_PAD - D_COMB - 1), jnp.float32)
    table_ref[...] = jnp.concatenate([normed, batchf, pad], axis=1)


def _make_sc_gather(b_total):
    info = plsc.get_sparse_core_info()
    nc, ns = info.num_cores, info.num_subcores
    nw = nc * ns
    b_per_w = b_total // nw
    mesh = plsc.VectorSubcoreMesh(core_axis_name="c", subcore_axis_name="s")

    @functools.partial(
        pl.kernel, mesh=mesh,
        out_type=jax.ShapeDtypeStruct((b_total, D_PAD), jnp.float32),
        scratch_types=[
            pltpu.VMEM((b_per_w,), jnp.int32),
            pltpu.VMEM((b_per_w, D_PAD), jnp.float32),
            pltpu.SemaphoreType.DMA,
        ],
    )
    def gather(table_hbm, idx_hbm, out_hbm, idx_v, rows_v, sem):
        wid = lax.axis_index("s") * nc + lax.axis_index("c")
        base = wid * b_per_w
        pltpu.sync_copy(idx_hbm.at[pl.ds(base, b_per_w)], idx_v)
        pltpu.async_copy(table_hbm.at[idx_v], rows_v, sem).wait()
        pltpu.sync_copy(rows_v, out_hbm.at[pl.ds(base, b_per_w)])

    return gather


def _dist_topk_kernel(q_ref, table_ref, cnrow_ref, brow_ref, col_ref, dist_s):
    q = q_ref[...]
    tbl = table_ref[...]
    lane = lax.broadcasted_iota(jnp.int32, (1, D_PAD), 1)
    qm = jnp.where(lane < D_COMB, q, 0.0)
    qn = jnp.sum(qm * qm, axis=1, keepdims=True)
    dot = lax.dot_general(qm, tbl, (((1,), (1,)), ((), ())),
                          preferred_element_type=jnp.float32)
    dist = (qn + cnrow_ref[...]) - 2.0 * dot
    bq = q[:, LANE_BATCH:LANE_BATCH + 1]
    dist = jnp.where(bq != brow_ref[...], 1e10, dist)
    dist_s[...] = dist

    bq_n = dist.shape[0]
    n = dist.shape[1]
    col_iota = lax.broadcasted_iota(jnp.int32, (1, n), 1)
    k_iota = lax.broadcasted_iota(jnp.int32, (bq_n, K), 1)

    def body(k, col_acc):
        d = dist_s[...]
        mn = jnp.min(d, axis=1, keepdims=True)
        idx = jnp.min(jnp.where(d == mn, col_iota, n), axis=1,
                      keepdims=True)
        dist_s[...] = jnp.where(col_iota == idx, jnp.float32(3e38), d)
        return jnp.where(k_iota == k, idx, col_acc)

    col_ref[...] = lax.fori_loop(0, K, body, jnp.zeros((bq_n, K), jnp.int32))


def kernel(x, pos, batch, focal_points, fa_w1, fa_b1, fa_w2, fa_b2,
           pa_w1, pa_b1, pa_w2, pa_b2):
    n = x.shape[0]
    qn_total = focal_points.shape[0]
    batchf = batch.astype(jnp.float32).reshape(n, 1)
    n_row_tiles = n // ROW_TILE

    weights = (fa_w1, fa_b1.reshape(1, ATT_DIM), fa_w2, fa_b2.reshape(1, 1),
               pa_w1, pa_b1.reshape(1, ATT_DIM), pa_w2, pa_b2.reshape(1, 1))
    w_specs = [pl.BlockSpec(w.shape, lambda i: (0, 0)) for w in weights]

    comb, cnt, ssum, sq = pl.pallas_call(
        _stage1_kernel,
        grid=(n_row_tiles,),
        in_specs=[
            pl.BlockSpec((ROW_TILE, 64), lambda i: (i, 0)),
            pl.BlockSpec((ROW_TILE, 3), lambda i: (i, 0)),
            pl.BlockSpec((ROW_TILE, 1), lambda i: (i, 0)),
        ] + w_specs,
        out_specs=(
            pl.BlockSpec((ROW_TILE, D_COMB), lambda i: (i, 0)),
            pl.BlockSpec((NUM_SEGMENTS, 1), lambda i: (0, 0)),
            pl.BlockSpec((NUM_SEGMENTS, D_COMB), lambda i: (0, 0)),
            pl.BlockSpec((NUM_SEGMENTS, D_COMB), lambda i: (0, 0)),
        ),
        out_shape=(
            jax.ShapeDtypeStruct((n, D_COMB), jnp.float32),
            jax.ShapeDtypeStruct((NUM_SEGMENTS, 1), jnp.float32),
            jax.ShapeDtypeStruct((NUM_SEGMENTS, D_COMB), jnp.float32),
            jax.ShapeDtypeStruct((NUM_SEGMENTS, D_COMB), jnp.float32),
        ),
    )(x, pos, batchf, *weights)

    table, cn = pl.pallas_call(
        _stage2_kernel,
        grid=(n_row_tiles,),
        in_specs=[
            pl.BlockSpec((ROW_TILE, D_COMB), lambda i: (i, 0)),
            pl.BlockSpec((ROW_TILE, 1), lambda i: (i, 0)),
            pl.BlockSpec((NUM_SEGMENTS, 1), lambda i: (0, 0)),
            pl.BlockSpec((NUM_SEGMENTS, D_COMB), lambda i: (0, 0)),
            pl.BlockSpec((NUM_SEGMENTS, D_COMB), lambda i: (0, 0)),
        ],
        out_specs=(
            pl.BlockSpec((ROW_TILE, D_PAD), lambda i: (i, 0)),
            pl.BlockSpec((ROW_TILE, 1), lambda i: (i, 0)),
        ),
        out_shape=(
            jax.ShapeDtypeStruct((n, D_PAD), jnp.float32),
            jax.ShapeDtypeStruct((n, 1), jnp.float32),
        ),
    )(comb, batchf, cnt, ssum, sq)

    q = _make_sc_gather(qn_total)(table, focal_points.astype(jnp.int32))

    cn_row = cn.reshape(1, n)
    b_row = batchf.reshape(1, n)

    grid = qn_total // Q_TILE
    col = pl.pallas_call(
        _dist_topk_kernel,
        grid=(grid,),
        in_specs=[
            pl.BlockSpec((Q_TILE, D_PAD), lambda i: (i, 0)),
            pl.BlockSpec((n, D_PAD), lambda i: (0, 0)),
            pl.BlockSpec((1, n), lambda i: (0, 0)),
            pl.BlockSpec((1, n), lambda i: (0, 0)),
        ],
        out_specs=pl.BlockSpec((Q_TILE, K), lambda i: (i, 0)),
        out_shape=jax.ShapeDtypeStruct((qn_total, K), jnp.int32),
        scratch_shapes=[pltpu.VMEM((Q_TILE, n), jnp.float32)],
    )(q, table, cn_row, b_row)

    row = jnp.repeat(jnp.arange(qn_total, dtype=jnp.int64), K)
    return (row, col.reshape(-1).astype(jnp.int64))
